# Initial kernel scaffold; baseline (speedup 1.0000x reference)
import functools

import jax
import jax.numpy as jnp
from jax import lax
from jax.experimental import pallas as pl
from jax.experimental.pallas import tpu as pltpu
from jax.experimental.pallas import tpu_sc as plsc

NC = 2
NS = 16
L = 16

f32 = jnp.float32
i32 = jnp.int32


def _tc_w2(comp, basis2):
    R, NB = comp.shape
    GH = basis2.shape[1]

    def body(c_ref, b_ref, o_ref):
        o_ref[...] = jnp.dot(c_ref[...], b_ref[...],
                             preferred_element_type=f32)

    return pl.pallas_call(
        body,
        out_shape=jax.ShapeDtypeStruct((R, GH), f32),
    )(comp, basis2)


def _tc_xw(x, w_full, bn):
    N, G = x.shape
    R = w_full.shape[0]
    H1 = w_full.shape[2]
    nb = N // bn

    def body(x_ref, w_ref, o_ref):
        o_ref[0] = jnp.dot(x_ref[...], w_ref[0], preferred_element_type=f32)

    return pl.pallas_call(
        body,
        grid=(R, nb),
        in_specs=[
            pl.BlockSpec((bn, G), lambda r, b: (b, 0)),
            pl.BlockSpec((1, G, H1), lambda r, b: (r, 0, 0)),
        ],
        out_specs=pl.BlockSpec((1, bn, H1), lambda r, b: (r, b, 0)),
        out_shape=jax.ShapeDtypeStruct((R, N, H1), f32),
    )(x, w_full)


def _tc_inv(cnt3):
    _, rows, cols = cnt3.shape

    def body(c_ref, o_ref):
        c = c_ref[0] + c_ref[1]
        o_ref[...] = 1.0 / jnp.maximum(c, 1.0)

    return pl.pallas_call(
        body,
        out_shape=jax.ShapeDtypeStruct((rows, cols), f32),
    )(cnt3)


def _tc_dense2(x, acc0, acc1, root, b1, Wk, bk, Wq, bq, Wv, bv, Wskip, bn):
    N, G = x.shape
    H1 = root.shape[1]
    H2 = Wk.shape[1]
    nb = N // bn

    def body(x_ref, a0_ref, a1_ref, rt_ref, b1_ref, wk_ref, bk_ref,
             wq_ref, bq_ref, wv_ref, bv_ref, ws_ref,
             k_ref, qv_ref, sk_ref):
        out1 = (jnp.dot(x_ref[...], rt_ref[...], preferred_element_type=f32)
                + b1_ref[...] + a0_ref[...] + a1_ref[...])
        k_ref[...] = jnp.dot(out1, wk_ref[...],
                             preferred_element_type=f32) + bk_ref[...]
        qv_ref[:, :H2] = jnp.dot(out1, wq_ref[...],
                                 preferred_element_type=f32) + bq_ref[...]
        qv_ref[:, H2:] = jnp.dot(out1, wv_ref[...],
                                 preferred_element_type=f32) + bv_ref[...]
        sk_ref[...] = jnp.dot(out1, ws_ref[...], preferred_element_type=f32)

    def full(s):
        return pl.BlockSpec(s, lambda b: tuple(0 for _ in s))

    blk = pl.BlockSpec((bn, H1), lambda b: (b, 0))
    return pl.pallas_call(
        body,
        grid=(nb,),
        in_specs=[
            pl.BlockSpec((bn, G), lambda b: (b, 0)),
            blk, blk,
            full((G, H1)), full((1, H1)),
            full((H1, H2)), full((1, H2)),
            full((H1, H2)), full((1, H2)),
            full((H1, H2)), full((1, H2)),
            full((H1, H2)),
        ],
        out_specs=[
            pl.BlockSpec((bn, H2), lambda b: (b, 0)),
            pl.BlockSpec((bn, 2 * H2), lambda b: (b, 0)),
            pl.BlockSpec((bn, H2), lambda b: (b, 0)),
        ],
        out_shape=[
            jax.ShapeDtypeStruct((N, H2), f32),
            jax.ShapeDtypeStruct((N, 2 * H2), f32),
            jax.ShapeDtypeStruct((N, H2), f32),
        ],
    )(x, acc0, acc1, root, b1, Wk, bk, Wq, bq, Wv, bv, Wskip)


def _tc_final(agg0, agg1, skip, b2, bn):
    N, H2 = skip.shape
    nb = N // bn
    blk = pl.BlockSpec((bn, H2), lambda b: (b, 0))

    def body(a0_ref, a1_ref, sk_ref, b2_ref, o_ref):
        o_ref[...] = a0_ref[...] + a1_ref[...] + sk_ref[...] + b2_ref[...]

    return pl.pallas_call(
        body,
        grid=(nb,),
        in_specs=[blk, blk, blk,
                  pl.BlockSpec((1, H2), lambda b: (0, 0))],
        out_specs=blk,
        out_shape=jax.ShapeDtypeStruct((N, H2), f32),
    )(agg0, agg1, skip, b2)


def _mesh():
    return plsc.VectorSubcoreMesh(core_axis_name="c", subcore_axis_name="s",
                                  num_cores=NC, num_subcores=NS)


def _sc_hist(edge_index, edge_type, N, R):
    E = edge_type.shape[0]
    EW = E // (NC * NS)
    CA = 2000
    nch = EW // CA
    T = R * N
    ZW = T // NS

    @functools.partial(
        pl.kernel,
        out_type=jax.ShapeDtypeStruct((NC, T), f32),
        mesh=_mesh(),
        scratch_types=[
            pltpu.VMEM((CA,), i32),
            pltpu.VMEM((CA,), i32),
            pltpu.VMEM((CA,), i32),
            pltpu.VMEM((CA,), f32),
            pltpu.VMEM((CA,), f32),
            pltpu.VMEM((ZW,), f32),
            pltpu.VMEM_SHARED((T,), f32),
            pltpu.SemaphoreType.DMA,
        ],
    )
    def hist(ei, et, out, t_v, d_v, idx_v, ones_v, zbuf, rbuf, cnt_sh, sem):
        c = lax.axis_index("c")
        s = lax.axis_index("s")
        w = c * NS + s

        def fill(i, _):
            ones_v[pl.ds(i * L, L)] = jnp.full((L,), 1.0, f32)
            zbuf[pl.ds(i * L, L)] = jnp.full((L,), 0.0, f32)
            return 0
        lax.fori_loop(0, CA // L, fill, 0)

        off = 0
        while off < ZW:
            n = min(CA, ZW - off)
            pltpu.sync_copy(zbuf.at[pl.ds(0, n)],
                            cnt_sh.at[pl.ds(s * ZW + off, n)])
            off += n
        plsc.subcore_barrier()

        def chunk(j, _):
            base = w * EW + j * CA
            pltpu.sync_copy(et.at[pl.ds(base, CA)], t_v)
            pltpu.sync_copy(ei.at[1, pl.ds(base, CA)], d_v)

            def cidx(i, _):
                sl = pl.ds(i * L, L)
                idx_v[sl] = t_v[sl] * N + d_v[sl]
                return 0
            lax.fori_loop(0, CA // L, cidx, 0)
            pltpu.sync_copy(ones_v, cnt_sh.at[idx_v], add=True)
            return 0
        lax.fori_loop(0, nch, chunk, 0)

        plsc.subcore_barrier()
        pltpu.sync_copy(cnt_sh.at[pl.ds(s * ZW, ZW)], rbuf)
        pltpu.sync_copy(rbuf, out.at[c, pl.ds(s * ZW, ZW)])

    return hist(edge_index, edge_type)


def _sc_rgcn(edge_index, edge_type, xw_tab, inv_tab, N):
    E = edge_type.shape[0]
    H = xw_tab.shape[1]
    EW = E // (NC * NS)
    CB = 400
    nch = EW // CB
    RW = N // NS
    nh = H // L

    @functools.partial(
        pl.kernel,
        out_type=jax.ShapeDtypeStruct((NC, N, H), f32),
        mesh=_mesh(),
        scratch_types=[
            pltpu.VMEM((CB,), i32),
            pltpu.VMEM((CB,), i32),
            pltpu.VMEM((CB,), i32),
            pltpu.VMEM((CB,), i32),
            pltpu.VMEM((CB,), i32),
            pltpu.VMEM((CB,), f32),
            pltpu.VMEM((CB, H), f32),
            pltpu.VMEM_SHARED((N, H), f32),
            pltpu.SemaphoreType.DMA,
        ],
    )
    def rgcn(ei, et, xw, invt, out, t_v, s_v, d_v, im_v, ic_v, inv_v,
             msg_v, acc_sh, sem):
        c = lax.axis_index("c")
        s = lax.axis_index("s")
        w = c * NS + s

        def zrow(i, _):
            for jj in range(nh):
                msg_v[i, pl.ds(jj * L, L)] = jnp.full((L,), 0.0, f32)
            return 0
        lax.fori_loop(0, CB, zrow, 0)

        off = 0
        while off < RW:
            n = min(CB, RW - off)
            pltpu.sync_copy(msg_v.at[pl.ds(0, n), :],
                            acc_sh.at[pl.ds(s * RW + off, n), :])
            off += n
        plsc.subcore_barrier()

        def chunk(j, _):
            base = w * EW + j * CB
            pltpu.sync_copy(et.at[pl.ds(base, CB)], t_v)
            pltpu.sync_copy(ei.at[0, pl.ds(base, CB)], s_v)
            pltpu.sync_copy(ei.at[1, pl.ds(base, CB)], d_v)

            def cidx(i, _):
                sl = pl.ds(i * L, L)
                tt = t_v[sl] * N
                im_v[sl] = tt + s_v[sl]
                ic_v[sl] = tt + d_v[sl]
                return 0
            lax.fori_loop(0, CB // L, cidx, 0)

            pltpu.async_copy(xw.at[im_v], msg_v, sem).wait()
            pltpu.async_copy(invt.at[ic_v], inv_v, sem).wait()

            def scale(i, _):
                sc = inv_v[i]
                for jj in range(nh):
                    sl = pl.ds(jj * L, L)
                    msg_v[i, sl] = msg_v[i, sl] * sc
                return 0
            lax.fori_loop(0, CB, scale, 0)

            pltpu.sync_copy(msg_v, acc_sh.at[d_v], add=True)
            return 0
        lax.fori_loop(0, nch, chunk, 0)

        plsc.subcore_barrier()
        off = 0
        while off < RW:
            n = min(CB, RW - off)
            pltpu.sync_copy(acc_sh.at[pl.ds(s * RW + off, n), :],
                            msg_v.at[pl.ds(0, n), :])
            pltpu.sync_copy(msg_v.at[pl.ds(0, n), :],
                            out.at[c, pl.ds(s * RW + off, n), :])
            off += n

    return rgcn(edge_index, edge_type, xw_tab, inv_tab)


def _sc_gated(edge_index, k_tab, qv_tab, N):
    E = edge_index.shape[1]
    H = k_tab.shape[1]
    EW = E // (NC * NS)
    CC = 200
    nch = EW // CC
    RW = N // NS
    nh = H // L

    @functools.partial(
        pl.kernel,
        out_type=jax.ShapeDtypeStruct((NC, N, H), f32),
        mesh=_mesh(),
        scratch_types=[
            pltpu.VMEM((CC,), i32),
            pltpu.VMEM((CC,), i32),
            pltpu.VMEM((CC, H), f32),
            pltpu.VMEM((CC, 2 * H), f32),
            pltpu.VMEM_SHARED((N, H), f32),
            pltpu.SemaphoreType.DMA,
        ],
    )
    def gated(ei, kt, qvt, out, s_v, d_v, kbuf, qvbuf, acc_sh, sem):
        c = lax.axis_index("c")
        s = lax.axis_index("s")
        w = c * NS + s

        def zrow(i, _):
            for jj in range(nh):
                kbuf[i, pl.ds(jj * L, L)] = jnp.full((L,), 0.0, f32)
            return 0
        lax.fori_loop(0, CC, zrow, 0)

        off = 0
        while off < RW:
            n = min(CC, RW - off)
            pltpu.sync_copy(kbuf.at[pl.ds(0, n), :],
                            acc_sh.at[pl.ds(s * RW + off, n), :])
            off += n
        plsc.subcore_barrier()

        def chunk(j, _):
            base = w * EW + j * CC
            pltpu.sync_copy(ei.at[0, pl.ds(base, CC)], s_v)
            pltpu.sync_copy(ei.at[1, pl.ds(base, CC)], d_v)

            pltpu.async_copy(kt.at[d_v], kbuf, sem).wait()
            pltpu.async_copy(qvt.at[s_v], qvbuf, sem).wait()

            def gate(i, _):
                for jj in range(nh):
                    sl = pl.ds(jj * L, L)
                    z = kbuf[i, sl] + qvbuf[i, sl]
                    sig = 1.0 / (1.0 + jnp.exp(-z))
                    kbuf[i, sl] = sig * qvbuf[i, pl.ds(H + jj * L, L)]
                return 0
            lax.fori_loop(0, CC, gate, 0)

            pltpu.sync_copy(kbuf, acc_sh.at[d_v], add=True)
            return 0
        lax.fori_loop(0, nch, chunk, 0)

        plsc.subcore_barrier()
        off = 0
        while off < RW:
            n = min(CC, RW - off)
            pltpu.sync_copy(acc_sh.at[pl.ds(s * RW + off, n), :],
                            kbuf.at[pl.ds(0, n), :])
            pltpu.sync_copy(kbuf.at[pl.ds(0, n), :],
                            out.at[c, pl.ds(s * RW + off, n), :])
            off += n

    return gated(edge_index, k_tab, qv_tab)


def kernel(node_features, edge_index, edge_norm, edge_type, comp, basis,
           root, bias1, Wk, bk, Wq, bq, Wv, bv, Wskip, bias2):
    del edge_norm
    N, G = node_features.shape
    R, NB = comp.shape
    H1 = root.shape[1]
    H2 = Wk.shape[1]
    bn = 1000

    w2 = _tc_w2(comp, basis.reshape(NB, G * H1))
    xw = _tc_xw(node_features, w2.reshape(R, G, H1), bn)
    xw_tab = xw.reshape(R * N, H1)

    cnt = _sc_hist(edge_index, edge_type, N, R)
    inv3 = _tc_inv(cnt.reshape(NC, (R * N) // G, G))
    inv_tab = inv3.reshape(R * N)

    accs = _sc_rgcn(edge_index, edge_type, xw_tab, inv_tab, N)

    k_tab, qv_tab, skip = _tc_dense2(
        node_features, accs[0], accs[1], root,
        bias1.reshape(1, H1), Wk, bk.reshape(1, H2), Wq, bq.reshape(1, H2),
        Wv, bv.reshape(1, H2), Wskip, bn)

    aggs = _sc_gated(edge_index, k_tab, qv_tab, N)

    return _tc_final(aggs[0], aggs[1], skip, bias2.reshape(1, H2), bn)


# speedup vs baseline: 6.1453x; 6.1453x over previous
import functools

import jax
import jax.numpy as jnp
from jax import lax
from jax.experimental import pallas as pl
from jax.experimental.pallas import tpu as pltpu
from jax.experimental.pallas import tpu_sc as plsc

NC = 2
NS = 16
L = 16

f32 = jnp.float32
i32 = jnp.int32


def _tc_w2(comp, basis2):
    R, NB = comp.shape
    GH = basis2.shape[1]

    def body(c_ref, b_ref, o_ref):
        o_ref[...] = jnp.dot(c_ref[...], b_ref[...],
                             preferred_element_type=f32)

    return pl.pallas_call(
        body,
        out_shape=jax.ShapeDtypeStruct((R, GH), f32),
    )(comp, basis2)


def _tc_xw(x, w_full, bn):
    N, G = x.shape
    R = w_full.shape[0]
    H1 = w_full.shape[2]
    nb = N // bn

    def body(x_ref, w_ref, o_ref):
        o_ref[0] = jnp.dot(x_ref[...], w_ref[0], preferred_element_type=f32)

    return pl.pallas_call(
        body,
        grid=(R, nb),
        in_specs=[
            pl.BlockSpec((bn, G), lambda r, b: (b, 0)),
            pl.BlockSpec((1, G, H1), lambda r, b: (r, 0, 0)),
        ],
        out_specs=pl.BlockSpec((1, bn, H1), lambda r, b: (r, b, 0)),
        out_shape=jax.ShapeDtypeStruct((R, N, H1), f32),
    )(x, w_full)


def _tc_inv(cnt3):
    _, rows, cols = cnt3.shape

    def body(c_ref, o_ref):
        c = c_ref[0] + c_ref[1]
        o_ref[...] = 1.0 / jnp.maximum(c, 1.0)

    return pl.pallas_call(
        body,
        out_shape=jax.ShapeDtypeStruct((rows, cols), f32),
    )(cnt3)


def _tc_dense2(x, acc0, acc1, root, b1, Wk, bk, Wq, bq, Wv, bv, Wskip, bn):
    N, G = x.shape
    H1 = root.shape[1]
    H2 = Wk.shape[1]
    nb = N // bn

    def body(x_ref, a0_ref, a1_ref, rt_ref, b1_ref, wk_ref, bk_ref,
             wq_ref, bq_ref, wv_ref, bv_ref, ws_ref,
             k_ref, qv_ref, sk_ref):
        out1 = (jnp.dot(x_ref[...], rt_ref[...], preferred_element_type=f32)
                + b1_ref[...] + a0_ref[...] + a1_ref[...])
        k_ref[...] = jnp.dot(out1, wk_ref[...],
                             preferred_element_type=f32) + bk_ref[...]
        qv_ref[:, :H2] = jnp.dot(out1, wq_ref[...],
                                 preferred_element_type=f32) + bq_ref[...]
        qv_ref[:, H2:] = jnp.dot(out1, wv_ref[...],
                                 preferred_element_type=f32) + bv_ref[...]
        sk_ref[...] = jnp.dot(out1, ws_ref[...], preferred_element_type=f32)

    def full(s):
        return pl.BlockSpec(s, lambda b: tuple(0 for _ in s))

    blk = pl.BlockSpec((bn, H1), lambda b: (b, 0))
    return pl.pallas_call(
        body,
        grid=(nb,),
        in_specs=[
            pl.BlockSpec((bn, G), lambda b: (b, 0)),
            blk, blk,
            full((G, H1)), full((1, H1)),
            full((H1, H2)), full((1, H2)),
            full((H1, H2)), full((1, H2)),
            full((H1, H2)), full((1, H2)),
            full((H1, H2)),
        ],
        out_specs=[
            pl.BlockSpec((bn, H2), lambda b: (b, 0)),
            pl.BlockSpec((bn, 2 * H2), lambda b: (b, 0)),
            pl.BlockSpec((bn, H2), lambda b: (b, 0)),
        ],
        out_shape=[
            jax.ShapeDtypeStruct((N, H2), f32),
            jax.ShapeDtypeStruct((N, 2 * H2), f32),
            jax.ShapeDtypeStruct((N, H2), f32),
        ],
    )(x, acc0, acc1, root, b1, Wk, bk, Wq, bq, Wv, bv, Wskip)


def _tc_final(agg0, agg1, skip, b2, bn):
    N, H2 = skip.shape
    nb = N // bn
    blk = pl.BlockSpec((bn, H2), lambda b: (b, 0))

    def body(a0_ref, a1_ref, sk_ref, b2_ref, o_ref):
        o_ref[...] = a0_ref[...] + a1_ref[...] + sk_ref[...] + b2_ref[...]

    return pl.pallas_call(
        body,
        grid=(nb,),
        in_specs=[blk, blk, blk,
                  pl.BlockSpec((1, H2), lambda b: (0, 0))],
        out_specs=blk,
        out_shape=jax.ShapeDtypeStruct((N, H2), f32),
    )(agg0, agg1, skip, b2)


def _mesh():
    return plsc.VectorSubcoreMesh(core_axis_name="c", subcore_axis_name="s",
                                  num_cores=NC, num_subcores=NS)


def _sc_hist(dst_arr, edge_type, N, R):
    E = edge_type.shape[0]
    EW = E // (NC * NS)
    CA = 80
    nch = EW // CA
    T = R * N
    ZW = T // NS

    @functools.partial(
        pl.kernel,
        out_type=jax.ShapeDtypeStruct((NC * T,), f32),
        mesh=_mesh(),
        scratch_types=[
            pltpu.VMEM((CA,), i32),
            pltpu.VMEM((CA,), i32),
            pltpu.VMEM((CA,), i32),
            pltpu.VMEM((CA,), f32),
            pltpu.VMEM((ZW,), f32),
            pltpu.VMEM_SHARED((T,), f32),
            pltpu.SemaphoreType.DMA,
        ],
    )
    def hist(dsta, et, out, t_v, d_v, idx_v, ones_v, zbuf, cnt_sh, sem):
        c = lax.axis_index("c")
        s = lax.axis_index("s")
        w = c * NS + s

        def fill1(i, _):
            ones_v[pl.ds(i * L, L)] = jnp.full((L,), 1.0, f32)
            return 0
        lax.fori_loop(0, CA // L, fill1, 0)

        def fill0(i, _):
            zbuf[pl.ds(i * L, L)] = jnp.full((L,), 0.0, f32)
            return 0
        lax.fori_loop(0, ZW // L, fill0, 0)

        pltpu.sync_copy(zbuf, cnt_sh.at[pl.ds(s * ZW, ZW)])
        plsc.subcore_barrier()

        def chunk(j, _):
            base = w * EW + j * CA
            pltpu.sync_copy(et.at[pl.ds(base, CA)], t_v)
            pltpu.sync_copy(dsta.at[pl.ds(base, CA)], d_v)

            def cidx(i, _):
                sl = pl.ds(i * L, L)
                idx_v[sl] = t_v[sl] * N + d_v[sl]
                return 0
            lax.fori_loop(0, CA // L, cidx, 0)
            pltpu.sync_copy(ones_v, cnt_sh.at[idx_v], add=True)
            return 0
        lax.fori_loop(0, nch, chunk, 0)

        plsc.subcore_barrier()
        pltpu.sync_copy(cnt_sh.at[pl.ds(s * ZW, ZW)], zbuf)
        pltpu.sync_copy(zbuf, out.at[pl.ds(c * T + s * ZW, ZW)])

    return hist(dst_arr, edge_type)


def _sc_rgcn(src_arr, dst_arr, edge_type, xw_tab, inv_tab, N, NP):
    E = edge_type.shape[0]
    H = xw_tab.shape[1]
    EW = E // (NC * NS)
    CB = 80
    nch = EW // CB
    RW = NP // NS
    nh = H // L

    @functools.partial(
        pl.kernel,
        out_type=jax.ShapeDtypeStruct((NC, NP, H), f32),
        mesh=_mesh(),
        scratch_types=[
            pltpu.VMEM((CB,), i32),
            pltpu.VMEM((CB,), i32),
            pltpu.VMEM((CB,), i32),
            pltpu.VMEM((CB,), i32),
            pltpu.VMEM((CB,), i32),
            pltpu.VMEM((CB,), f32),
            pltpu.VMEM((CB, H), f32),
            pltpu.VMEM_SHARED((NP, H), f32),
            pltpu.SemaphoreType.DMA,
        ],
    )
    def rgcn(srca, dsta, et, xw, invt, out, t_v, s_v, d_v, im_v, ic_v, inv_v,
             msg_v, acc_sh, sem):
        c = lax.axis_index("c")
        s = lax.axis_index("s")
        w = c * NS + s

        def zrow(i, _):
            for jj in range(nh):
                msg_v[i, pl.ds(jj * L, L)] = jnp.full((L,), 0.0, f32)
            return 0
        lax.fori_loop(0, CB, zrow, 0)

        for off in range(0, RW, CB):
            pltpu.sync_copy(msg_v, acc_sh.at[pl.ds(s * RW + off, CB), :])
        plsc.subcore_barrier()

        def chunk(j, _):
            base = w * EW + j * CB
            pltpu.sync_copy(et.at[pl.ds(base, CB)], t_v)
            pltpu.sync_copy(srca.at[pl.ds(base, CB)], s_v)
            pltpu.sync_copy(dsta.at[pl.ds(base, CB)], d_v)

            def cidx(i, _):
                sl = pl.ds(i * L, L)
                tt = t_v[sl] * N
                im_v[sl] = tt + s_v[sl]
                ic_v[sl] = tt + d_v[sl]
                return 0
            lax.fori_loop(0, CB // L, cidx, 0)

            pltpu.async_copy(xw.at[im_v], msg_v, sem).wait()
            pltpu.async_copy(invt.at[ic_v], inv_v, sem).wait()

            def scale(g, _):
                iv = inv_v[pl.ds(g * L, L)]
                for e in range(L):
                    i = g * L + e
                    sc = iv[e]
                    for jj in range(nh):
                        sl = pl.ds(jj * L, L)
                        msg_v[i, sl] = msg_v[i, sl] * sc
                return 0
            lax.fori_loop(0, CB // L, scale, 0)

            pltpu.sync_copy(msg_v, acc_sh.at[d_v], add=True)
            return 0
        lax.fori_loop(0, nch, chunk, 0)

        plsc.subcore_barrier()
        for off in range(0, RW, CB):
            pltpu.sync_copy(acc_sh.at[pl.ds(s * RW + off, CB), :], msg_v)
            pltpu.sync_copy(msg_v, out.at[c, pl.ds(s * RW + off, CB), :])

    return rgcn(src_arr, dst_arr, edge_type, xw_tab, inv_tab)


def _sc_gated(src_arr, dst_arr, k_tab, qv_tab, NP):
    E = src_arr.shape[0]
    H = k_tab.shape[1]
    EW = E // (NC * NS)
    CC = 80
    nch = EW // CC
    RW = NP // NS
    nh = H // L

    @functools.partial(
        pl.kernel,
        out_type=jax.ShapeDtypeStruct((NC, NP, H), f32),
        mesh=_mesh(),
        scratch_types=[
            pltpu.VMEM((CC,), i32),
            pltpu.VMEM((CC,), i32),
            pltpu.VMEM((CC, H), f32),
            pltpu.VMEM((CC, 2 * H), f32),
            pltpu.VMEM_SHARED((NP, H), f32),
            pltpu.SemaphoreType.DMA,
        ],
    )
    def gated(srca, dsta, kt, qvt, out, s_v, d_v, kbuf, qvbuf, acc_sh, sem):
        c = lax.axis_index("c")
        s = lax.axis_index("s")
        w = c * NS + s

        def zrow(i, _):
            for jj in range(nh):
                kbuf[i, pl.ds(jj * L, L)] = jnp.full((L,), 0.0, f32)
            return 0
        lax.fori_loop(0, CC, zrow, 0)

        for off in range(0, RW, CC):
            pltpu.sync_copy(kbuf, acc_sh.at[pl.ds(s * RW + off, CC), :])
        plsc.subcore_barrier()

        def chunk(j, _):
            base = w * EW + j * CC
            pltpu.sync_copy(srca.at[pl.ds(base, CC)], s_v)
            pltpu.sync_copy(dsta.at[pl.ds(base, CC)], d_v)

            pltpu.async_copy(kt.at[d_v], kbuf, sem).wait()
            pltpu.async_copy(qvt.at[s_v], qvbuf, sem).wait()

            def gate(i, _):
                for jj in range(nh):
                    sl = pl.ds(jj * L, L)
                    z = kbuf[i, sl] + qvbuf[i, sl]
                    sig = 1.0 / (1.0 + jnp.exp(-z))
                    kbuf[i, sl] = sig * qvbuf[i, pl.ds(H + jj * L, L)]
                return 0
            lax.fori_loop(0, CC, gate, 0)

            pltpu.sync_copy(kbuf, acc_sh.at[d_v], add=True)
            return 0
        lax.fori_loop(0, nch, chunk, 0)

        plsc.subcore_barrier()
        for off in range(0, RW, CC):
            pltpu.sync_copy(acc_sh.at[pl.ds(s * RW + off, CC), :], kbuf)
            pltpu.sync_copy(kbuf, out.at[c, pl.ds(s * RW + off, CC), :])

    return gated(src_arr, dst_arr, k_tab, qv_tab)


def kernel(node_features, edge_index, edge_norm, edge_type, comp, basis,
           root, bias1, Wk, bk, Wq, bq, Wv, bv, Wskip, bias2):
    del edge_norm
    N, G = node_features.shape
    R, NB = comp.shape
    H1 = root.shape[1]
    H2 = Wk.shape[1]
    bn = 1000
    NP = 10240

    w2 = _tc_w2(comp, basis.reshape(NB, G * H1))
    xw = _tc_xw(node_features, w2.reshape(R, G, H1), bn)
    xw_tab = xw.reshape(R * N, H1)

    src_arr = edge_index[0]
    dst_arr = edge_index[1]

    cnt = _sc_hist(dst_arr, edge_type, N, R)
    inv3 = _tc_inv(cnt.reshape(NC, (R * N) // G, G))
    inv_tab = inv3.reshape(R * N)

    accs = _sc_rgcn(src_arr, dst_arr, edge_type, xw_tab, inv_tab, N, NP)

    k_tab, qv_tab, skip = _tc_dense2(
        node_features, accs[0, :N], accs[1, :N], root,
        bias1.reshape(1, H1), Wk, bk.reshape(1, H2), Wq, bq.reshape(1, H2),
        Wv, bv.reshape(1, H2), Wskip, bn)

    aggs = _sc_gated(src_arr, dst_arr, k_tab, qv_tab, NP)

    return _tc_final(aggs[0, :N], aggs[1, :N], skip, bias2.reshape(1, H2), bn)


# speedup vs baseline: 6.5187x; 1.0608x over previous
import functools

import jax
import jax.numpy as jnp
from jax import lax
from jax.experimental import pallas as pl
from jax.experimental.pallas import tpu as pltpu
from jax.experimental.pallas import tpu_sc as plsc

NC = 2
NS = 16
L = 16

f32 = jnp.float32
i32 = jnp.int32


def _tc_w2(comp, basis2):
    R, NB = comp.shape
    GH = basis2.shape[1]

    def body(c_ref, b_ref, o_ref):
        o_ref[...] = jnp.dot(c_ref[...], b_ref[...],
                             preferred_element_type=f32)

    return pl.pallas_call(
        body,
        out_shape=jax.ShapeDtypeStruct((R, GH), f32),
    )(comp, basis2)


def _tc_xw(x, w_full, bn):
    N, G = x.shape
    R = w_full.shape[0]
    H1 = w_full.shape[2]
    nb = N // bn

    def body(x_ref, w_ref, o_ref):
        o_ref[0] = jnp.dot(x_ref[...], w_ref[0], preferred_element_type=f32)

    return pl.pallas_call(
        body,
        grid=(R, nb),
        in_specs=[
            pl.BlockSpec((bn, G), lambda r, b: (b, 0)),
            pl.BlockSpec((1, G, H1), lambda r, b: (r, 0, 0)),
        ],
        out_specs=pl.BlockSpec((1, bn, H1), lambda r, b: (r, b, 0)),
        out_shape=jax.ShapeDtypeStruct((R, N, H1), f32),
    )(x, w_full)


def _tc_inv(cnt3):
    _, rows, cols = cnt3.shape

    def body(c_ref, o_ref):
        c = c_ref[0] + c_ref[1]
        o_ref[...] = 1.0 / jnp.maximum(c, 1.0)

    return pl.pallas_call(
        body,
        out_shape=jax.ShapeDtypeStruct((rows, cols), f32),
    )(cnt3)


def _tc_dense2(x, acc0, acc1, root, b1, Wk, bk, Wq, bq, Wv, bv, Wskip, bn):
    N, G = x.shape
    H1 = root.shape[1]
    H2 = Wk.shape[1]
    nb = N // bn

    def body(x_ref, a0_ref, a1_ref, rt_ref, b1_ref, wk_ref, bk_ref,
             wq_ref, bq_ref, wv_ref, bv_ref, ws_ref,
             k_ref, qv_ref, sk_ref):
        out1 = (jnp.dot(x_ref[...], rt_ref[...], preferred_element_type=f32)
                + b1_ref[...] + a0_ref[...] + a1_ref[...])
        k_ref[...] = jnp.dot(out1, wk_ref[...],
                             preferred_element_type=f32) + bk_ref[...]
        qv_ref[:, :H2] = jnp.dot(out1, wq_ref[...],
                                 preferred_element_type=f32) + bq_ref[...]
        qv_ref[:, H2:] = jnp.dot(out1, wv_ref[...],
                                 preferred_element_type=f32) + bv_ref[...]
        sk_ref[...] = jnp.dot(out1, ws_ref[...], preferred_element_type=f32)

    def full(s):
        return pl.BlockSpec(s, lambda b: tuple(0 for _ in s))

    blk = pl.BlockSpec((bn, H1), lambda b: (b, 0))
    return pl.pallas_call(
        body,
        grid=(nb,),
        in_specs=[
            pl.BlockSpec((bn, G), lambda b: (b, 0)),
            blk, blk,
            full((G, H1)), full((1, H1)),
            full((H1, H2)), full((1, H2)),
            full((H1, H2)), full((1, H2)),
            full((H1, H2)), full((1, H2)),
            full((H1, H2)),
        ],
        out_specs=[
            pl.BlockSpec((bn, H2), lambda b: (b, 0)),
            pl.BlockSpec((bn, 2 * H2), lambda b: (b, 0)),
            pl.BlockSpec((bn, H2), lambda b: (b, 0)),
        ],
        out_shape=[
            jax.ShapeDtypeStruct((N, H2), f32),
            jax.ShapeDtypeStruct((N, 2 * H2), f32),
            jax.ShapeDtypeStruct((N, H2), f32),
        ],
    )(x, acc0, acc1, root, b1, Wk, bk, Wq, bq, Wv, bv, Wskip)


def _tc_final(agg0, agg1, skip, b2, bn):
    N, H2 = skip.shape
    nb = N // bn
    blk = pl.BlockSpec((bn, H2), lambda b: (b, 0))

    def body(a0_ref, a1_ref, sk_ref, b2_ref, o_ref):
        o_ref[...] = a0_ref[...] + a1_ref[...] + sk_ref[...] + b2_ref[...]

    return pl.pallas_call(
        body,
        grid=(nb,),
        in_specs=[blk, blk, blk,
                  pl.BlockSpec((1, H2), lambda b: (0, 0))],
        out_specs=blk,
        out_shape=jax.ShapeDtypeStruct((N, H2), f32),
    )(agg0, agg1, skip, b2)


def _mesh():
    return plsc.VectorSubcoreMesh(core_axis_name="c", subcore_axis_name="s",
                                  num_cores=NC, num_subcores=NS)


def _sc_hist(dst_arr, edge_type, N, R):
    E = edge_type.shape[0]
    EW = E // (NC * NS)
    CA = 80
    nch = EW // CA
    T = R * N
    ZW = T // NS

    @functools.partial(
        pl.kernel,
        out_type=jax.ShapeDtypeStruct((NC * T,), f32),
        mesh=_mesh(),
        scratch_types=[
            pltpu.VMEM((CA,), i32),
            pltpu.VMEM((CA,), i32),
            pltpu.VMEM((CA,), i32),
            pltpu.VMEM((CA,), f32),
            pltpu.VMEM((ZW,), f32),
            pltpu.VMEM_SHARED((T,), f32),
            pltpu.SemaphoreType.DMA,
        ],
    )
    def hist(dsta, et, out, t_v, d_v, idx_v, ones_v, zbuf, cnt_sh, sem):
        c = lax.axis_index("c")
        s = lax.axis_index("s")
        w = c * NS + s

        def fill1(i, _):
            ones_v[pl.ds(i * L, L)] = jnp.full((L,), 1.0, f32)
            return 0
        lax.fori_loop(0, CA // L, fill1, 0)

        def fill0(i, _):
            zbuf[pl.ds(i * L, L)] = jnp.full((L,), 0.0, f32)
            return 0
        lax.fori_loop(0, ZW // L, fill0, 0)

        pltpu.sync_copy(zbuf, cnt_sh.at[pl.ds(s * ZW, ZW)])
        plsc.subcore_barrier()

        def chunk(j, _):
            base = w * EW + j * CA
            pltpu.sync_copy(et.at[pl.ds(base, CA)], t_v)
            pltpu.sync_copy(dsta.at[pl.ds(base, CA)], d_v)

            def cidx(i, _):
                sl = pl.ds(i * L, L)
                idx_v[sl] = t_v[sl] * N + d_v[sl]
                return 0
            lax.fori_loop(0, CA // L, cidx, 0)
            pltpu.sync_copy(ones_v, cnt_sh.at[idx_v], add=True)
            return 0
        lax.fori_loop(0, nch, chunk, 0)

        plsc.subcore_barrier()
        pltpu.sync_copy(cnt_sh.at[pl.ds(s * ZW, ZW)], zbuf)
        pltpu.sync_copy(zbuf, out.at[pl.ds(c * T + s * ZW, ZW)])

    return hist(dst_arr, edge_type)


def _sc_rgcn(src_arr, dst_arr, edge_type, xw_tab, inv_tab, N, NP):
    E = edge_type.shape[0]
    H = xw_tab.shape[1]
    EW = E // (NC * NS)
    CB = 80
    nch = EW // CB
    RW = NP // NS
    nh = H // L

    @functools.partial(
        pl.kernel,
        out_type=jax.ShapeDtypeStruct((NC, NP, H), f32),
        mesh=_mesh(),
        scratch_types=[
            pltpu.VMEM((CB,), i32),
            pltpu.VMEM((CB,), i32),
            pltpu.VMEM((CB,), i32),
            pltpu.VMEM((CB,), i32),
            pltpu.VMEM((CB,), i32),
            pltpu.VMEM((CB,), f32),
            pltpu.VMEM((CB, H), f32),
            pltpu.VMEM_SHARED((NP, H), f32),
            pltpu.SemaphoreType.DMA,
        ],
    )
    def rgcn(srca, dsta, et, xw, invt, out, t_v, s_v, d_v, im_v, ic_v, inv_v,
             msg_v, acc_sh, sem):
        c = lax.axis_index("c")
        s = lax.axis_index("s")
        w = c * NS + s

        def zrow(i, _):
            for jj in range(nh):
                msg_v[i, pl.ds(jj * L, L)] = jnp.full((L,), 0.0, f32)
            return 0
        lax.fori_loop(0, CB, zrow, 0)

        for off in range(0, RW, CB):
            pltpu.sync_copy(msg_v, acc_sh.at[pl.ds(s * RW + off, CB), :])
        plsc.subcore_barrier()

        def chunk(j, _):
            base = w * EW + j * CB
            pltpu.sync_copy(et.at[pl.ds(base, CB)], t_v)
            pltpu.sync_copy(srca.at[pl.ds(base, CB)], s_v)
            pltpu.sync_copy(dsta.at[pl.ds(base, CB)], d_v)

            def cidx(i, _):
                sl = pl.ds(i * L, L)
                tt = t_v[sl] * N
                im_v[sl] = tt + s_v[sl]
                ic_v[sl] = tt + d_v[sl]
                return 0
            lax.fori_loop(0, CB // L, cidx, 0)

            pltpu.async_copy(xw.at[im_v], msg_v, sem).wait()
            pltpu.async_copy(invt.at[ic_v], inv_v, sem).wait()

            def scale(g, _):
                iv = inv_v[pl.ds(g * L, L)]
                for e in range(L):
                    i = g * L + e
                    sc = iv[e]
                    for jj in range(nh):
                        sl = pl.ds(jj * L, L)
                        msg_v[i, sl] = msg_v[i, sl] * sc
                return 0
            lax.fori_loop(0, CB // L, scale, 0)

            pltpu.sync_copy(msg_v, acc_sh.at[d_v], add=True)
            return 0
        lax.fori_loop(0, nch, chunk, 0)

        plsc.subcore_barrier()
        for off in range(0, RW, CB):
            pltpu.sync_copy(acc_sh.at[pl.ds(s * RW + off, CB), :], msg_v)
            pltpu.sync_copy(msg_v, out.at[c, pl.ds(s * RW + off, CB), :])

    return rgcn(src_arr, dst_arr, edge_type, xw_tab, inv_tab)


def _sc_gated(src_arr, dst_arr, k_tab, qv_tab, NP):
    E = src_arr.shape[0]
    H = k_tab.shape[1]
    EW = E // (NC * NS)
    CC = 40
    DEPTH = 2
    nit = EW // (CC * DEPTH)
    RW = NP // NS
    CR = CC
    nh = H // L

    @functools.partial(
        pl.kernel,
        out_type=jax.ShapeDtypeStruct((NC, NP, H), f32),
        mesh=_mesh(),
        scratch_types=(
            [pltpu.VMEM((CC,), i32) for _ in range(DEPTH)] +
            [pltpu.VMEM((CC,), i32) for _ in range(DEPTH)] +
            [pltpu.VMEM((CC, H), f32) for _ in range(DEPTH)] +
            [pltpu.VMEM((CC, 2 * H), f32) for _ in range(DEPTH)] +
            [pltpu.VMEM_SHARED((NP, H), f32)] +
            [pltpu.SemaphoreType.DMA for _ in range(2 * DEPTH)]
        ),
    )
    def gated(srca, dsta, kt, qvt, out, *refs):
        s_v = refs[:DEPTH]
        d_v = refs[DEPTH:2 * DEPTH]
        kbuf = refs[2 * DEPTH:3 * DEPTH]
        qvbuf = refs[3 * DEPTH:4 * DEPTH]
        iobuf = kbuf[0]
        acc_sh = refs[4 * DEPTH]
        sems = refs[4 * DEPTH + 1:]

        c = lax.axis_index("c")
        s = lax.axis_index("s")
        w = c * NS + s

        def zrow(i, _):
            for jj in range(nh):
                iobuf[i, pl.ds(jj * L, L)] = jnp.full((L,), 0.0, f32)
            return 0
        lax.fori_loop(0, CR, zrow, 0)

        for off in range(0, RW, CR):
            pltpu.sync_copy(iobuf, acc_sh.at[pl.ds(s * RW + off, CR), :])
        plsc.subcore_barrier()

        def body(t, _):
            base0 = w * EW + t * (CC * DEPTH)
            cps = []
            for k in range(DEPTH):
                base = base0 + k * CC
                pltpu.sync_copy(srca.at[pl.ds(base, CC)], s_v[k])
                pltpu.sync_copy(dsta.at[pl.ds(base, CC)], d_v[k])
                ck = pltpu.async_copy(kt.at[d_v[k]], kbuf[k], sems[2 * k])
                cq = pltpu.async_copy(qvt.at[s_v[k]], qvbuf[k], sems[2 * k + 1])
                cps.append((ck, cq))
            for k in range(DEPTH):
                cps[k][0].wait()
                cps[k][1].wait()

                def gate(i, _):
                    for jj in range(nh):
                        sl = pl.ds(jj * L, L)
                        z = kbuf[k][i, sl] + qvbuf[k][i, sl]
                        sig = 1.0 / (1.0 + jnp.exp(-z))
                        kbuf[k][i, sl] = sig * qvbuf[k][i, pl.ds(H + jj * L, L)]
                    return 0
                lax.fori_loop(0, CC, gate, 0)

                pltpu.sync_copy(kbuf[k], acc_sh.at[d_v[k]], add=True)
            return 0
        lax.fori_loop(0, nit, body, 0)

        plsc.subcore_barrier()
        for off in range(0, RW, CR):
            pltpu.sync_copy(acc_sh.at[pl.ds(s * RW + off, CR), :], iobuf)
            pltpu.sync_copy(iobuf, out.at[c, pl.ds(s * RW + off, CR), :])

    return gated(src_arr, dst_arr, k_tab, qv_tab)


def kernel(node_features, edge_index, edge_norm, edge_type, comp, basis,
           root, bias1, Wk, bk, Wq, bq, Wv, bv, Wskip, bias2):
    del edge_norm
    N, G = node_features.shape
    R, NB = comp.shape
    H1 = root.shape[1]
    H2 = Wk.shape[1]
    bn = 1000
    NP = 10240

    w2 = _tc_w2(comp, basis.reshape(NB, G * H1))
    xw = _tc_xw(node_features, w2.reshape(R, G, H1), bn)
    xw_tab = xw.reshape(R * N, H1)

    src_arr = edge_index[0]
    dst_arr = edge_index[1]

    cnt = _sc_hist(dst_arr, edge_type, N, R)
    inv3 = _tc_inv(cnt.reshape(NC, (R * N) // G, G))
    inv_tab = inv3.reshape(R * N)

    accs = _sc_rgcn(src_arr, dst_arr, edge_type, xw_tab, inv_tab, N, NP)

    k_tab, qv_tab, skip = _tc_dense2(
        node_features, accs[0, :N], accs[1, :N], root,
        bias1.reshape(1, H1), Wk, bk.reshape(1, H2), Wq, bq.reshape(1, H2),
        Wv, bv.reshape(1, H2), Wskip, bn)

    aggs = _sc_gated(src_arr, dst_arr, k_tab, qv_tab, NP)

    return _tc_final(aggs[0, :N], aggs[1, :N], skip, bias2.reshape(1, H2), bn)


# speedup vs baseline: 7.3381x; 1.1257x over previous
import functools

import jax
import jax.numpy as jnp
from jax import lax
from jax.experimental import pallas as pl
from jax.experimental.pallas import tpu as pltpu
from jax.experimental.pallas import tpu_sc as plsc

NC = 2
NS = 16
L = 16

f32 = jnp.float32
i32 = jnp.int32


def _tc_w2(comp, basis2):
    R, NB = comp.shape
    GH = basis2.shape[1]

    def body(c_ref, b_ref, o_ref):
        o_ref[...] = jnp.dot(c_ref[...], b_ref[...],
                             preferred_element_type=f32)

    return pl.pallas_call(
        body,
        out_shape=jax.ShapeDtypeStruct((R, GH), f32),
    )(comp, basis2)


def _tc_xw(x, w_full, bn):
    N, G = x.shape
    R = w_full.shape[0]
    H1 = w_full.shape[2]
    nb = N // bn

    def body(x_ref, w_ref, o_ref):
        o_ref[0] = jnp.dot(x_ref[...], w_ref[0], preferred_element_type=f32)

    return pl.pallas_call(
        body,
        grid=(R, nb),
        in_specs=[
            pl.BlockSpec((bn, G), lambda r, b: (b, 0)),
            pl.BlockSpec((1, G, H1), lambda r, b: (r, 0, 0)),
        ],
        out_specs=pl.BlockSpec((1, bn, H1), lambda r, b: (r, b, 0)),
        out_shape=jax.ShapeDtypeStruct((R, N, H1), f32),
    )(x, w_full)


def _tc_inv(cnt3):
    _, rows, cols = cnt3.shape

    def body(c_ref, o_ref):
        c = c_ref[0] + c_ref[1]
        o_ref[...] = 1.0 / jnp.maximum(c, 1.0)

    return pl.pallas_call(
        body,
        out_shape=jax.ShapeDtypeStruct((rows, cols), f32),
    )(cnt3)


def _tc_dense2(x, acc0, acc1, root, b1, Wk, bk, Wq, bq, Wv, bv, Wskip, bn):
    N, G = x.shape
    H1 = root.shape[1]
    H2 = Wk.shape[1]
    nb = N // bn

    def body(x_ref, a0_ref, a1_ref, rt_ref, b1_ref, wk_ref, bk_ref,
             wq_ref, bq_ref, wv_ref, bv_ref, ws_ref,
             k_ref, qv_ref, sk_ref):
        out1 = (jnp.dot(x_ref[...], rt_ref[...], preferred_element_type=f32)
                + b1_ref[...] + a0_ref[...] + a1_ref[...])
        k_ref[...] = jnp.dot(out1, wk_ref[...],
                             preferred_element_type=f32) + bk_ref[...]
        qv_ref[:, :H2] = jnp.dot(out1, wq_ref[...],
                                 preferred_element_type=f32) + bq_ref[...]
        qv_ref[:, H2:] = jnp.dot(out1, wv_ref[...],
                                 preferred_element_type=f32) + bv_ref[...]
        sk_ref[...] = jnp.dot(out1, ws_ref[...], preferred_element_type=f32)

    def full(s):
        return pl.BlockSpec(s, lambda b: tuple(0 for _ in s))

    blk = pl.BlockSpec((bn, H1), lambda b: (b, 0))
    return pl.pallas_call(
        body,
        grid=(nb,),
        in_specs=[
            pl.BlockSpec((bn, G), lambda b: (b, 0)),
            blk, blk,
            full((G, H1)), full((1, H1)),
            full((H1, H2)), full((1, H2)),
            full((H1, H2)), full((1, H2)),
            full((H1, H2)), full((1, H2)),
            full((H1, H2)),
        ],
        out_specs=[
            pl.BlockSpec((bn, H2), lambda b: (b, 0)),
            pl.BlockSpec((bn, 2 * H2), lambda b: (b, 0)),
            pl.BlockSpec((bn, H2), lambda b: (b, 0)),
        ],
        out_shape=[
            jax.ShapeDtypeStruct((N, H2), f32),
            jax.ShapeDtypeStruct((N, 2 * H2), f32),
            jax.ShapeDtypeStruct((N, H2), f32),
        ],
    )(x, acc0, acc1, root, b1, Wk, bk, Wq, bq, Wv, bv, Wskip)


def _tc_final(agg0, agg1, skip, b2, bn):
    N, H2 = skip.shape
    nb = N // bn
    blk = pl.BlockSpec((bn, H2), lambda b: (b, 0))

    def body(a0_ref, a1_ref, sk_ref, b2_ref, o_ref):
        o_ref[...] = a0_ref[...] + a1_ref[...] + sk_ref[...] + b2_ref[...]

    return pl.pallas_call(
        body,
        grid=(nb,),
        in_specs=[blk, blk, blk,
                  pl.BlockSpec((1, H2), lambda b: (0, 0))],
        out_specs=blk,
        out_shape=jax.ShapeDtypeStruct((N, H2), f32),
    )(agg0, agg1, skip, b2)


def _mesh():
    return plsc.VectorSubcoreMesh(core_axis_name="c", subcore_axis_name="s",
                                  num_cores=NC, num_subcores=NS)


def _sc_hist(dst_arr, edge_type, N, R):
    E = edge_type.shape[0]
    EW = E // (NC * NS)
    CA = 80
    nch = EW // CA
    T = R * N
    ZW = T // NS

    @functools.partial(
        pl.kernel,
        out_type=jax.ShapeDtypeStruct((NC * T,), f32),
        mesh=_mesh(),
        scratch_types=[
            pltpu.VMEM((EW,), i32),
            pltpu.VMEM((EW,), i32),
            pltpu.VMEM((CA,), i32),
            pltpu.VMEM((CA,), f32),
            pltpu.VMEM((ZW,), f32),
            pltpu.VMEM_SHARED((T,), f32),
            pltpu.SemaphoreType.DMA,
        ],
    )
    def hist(dsta, et, out, t_all, d_all, idx_v, ones_v, zbuf, cnt_sh, sem):
        c = lax.axis_index("c")
        s = lax.axis_index("s")
        w = c * NS + s

        pltpu.sync_copy(et.at[pl.ds(w * EW, EW)], t_all)
        pltpu.sync_copy(dsta.at[pl.ds(w * EW, EW)], d_all)

        def fill1(i, _):
            ones_v[pl.ds(i * L, L)] = jnp.full((L,), 1.0, f32)
            return 0
        lax.fori_loop(0, CA // L, fill1, 0)

        def fill0(i, _):
            zbuf[pl.ds(i * L, L)] = jnp.full((L,), 0.0, f32)
            return 0
        lax.fori_loop(0, ZW // L, fill0, 0)

        pltpu.sync_copy(zbuf, cnt_sh.at[pl.ds(s * ZW, ZW)])
        plsc.subcore_barrier()

        def chunk(j, _):
            base = j * CA

            def cidx(i, _):
                sl = pl.ds(base + i * L, L)
                idx_v[pl.ds(i * L, L)] = t_all[sl] * N + d_all[sl]
                return 0
            lax.fori_loop(0, CA // L, cidx, 0)
            pltpu.sync_copy(ones_v, cnt_sh.at[idx_v], add=True)
            return 0
        lax.fori_loop(0, nch, chunk, 0)

        plsc.subcore_barrier()
        pltpu.sync_copy(cnt_sh.at[pl.ds(s * ZW, ZW)], zbuf)
        pltpu.sync_copy(zbuf, out.at[pl.ds(c * T + s * ZW, ZW)])

    return hist(dst_arr, edge_type)


def _sc_rgcn(src_arr, dst_arr, edge_type, xw_tab, inv_tab, N, NP):
    E = edge_type.shape[0]
    H = xw_tab.shape[1]
    EW = E // (NC * NS)
    CB = 80
    nch = EW // CB
    RW = NP // NS
    nh = H // L

    @functools.partial(
        pl.kernel,
        out_type=jax.ShapeDtypeStruct((NC, NP, H), f32),
        mesh=_mesh(),
        scratch_types=[
            pltpu.VMEM((EW,), i32),
            pltpu.VMEM((EW,), i32),
            pltpu.VMEM((EW,), i32),
            pltpu.VMEM((CB,), i32),
            pltpu.VMEM((CB,), i32),
            pltpu.VMEM((CB,), i32),
            pltpu.VMEM((CB,), f32),
            pltpu.VMEM((CB, H), f32),
            pltpu.VMEM_SHARED((NP, H), f32),
            pltpu.SemaphoreType.DMA,
        ],
    )
    def rgcn(srca, dsta, et, xw, invt, out, t_all, s_all, d_all, im_v, ic_v,
             sd_v, inv_v, msg_v, acc_sh, sem):
        c = lax.axis_index("c")
        s = lax.axis_index("s")
        w = c * NS + s

        pltpu.sync_copy(et.at[pl.ds(w * EW, EW)], t_all)
        pltpu.sync_copy(srca.at[pl.ds(w * EW, EW)], s_all)
        pltpu.sync_copy(dsta.at[pl.ds(w * EW, EW)], d_all)

        def zrow(i, _):
            for jj in range(nh):
                msg_v[i, pl.ds(jj * L, L)] = jnp.full((L,), 0.0, f32)
            return 0
        lax.fori_loop(0, CB, zrow, 0)

        for off in range(0, RW, CB):
            pltpu.sync_copy(msg_v, acc_sh.at[pl.ds(s * RW + off, CB), :])
        plsc.subcore_barrier()

        def chunk(j, _):
            base = j * CB

            def cidx(i, _):
                sl = pl.ds(base + i * L, L)
                ol = pl.ds(i * L, L)
                tt = t_all[sl] * N
                im_v[ol] = tt + s_all[sl]
                ic_v[ol] = tt + d_all[sl]
                sd_v[ol] = d_all[sl]
                return 0
            lax.fori_loop(0, CB // L, cidx, 0)

            cm = pltpu.async_copy(xw.at[im_v], msg_v, sem)
            ci = pltpu.async_copy(invt.at[ic_v], inv_v, sem)
            cm.wait()
            ci.wait()

            def scale(g, _):
                iv = inv_v[pl.ds(g * L, L)]
                for e in range(L):
                    i = g * L + e
                    sc = iv[e]
                    for jj in range(nh):
                        sl = pl.ds(jj * L, L)
                        msg_v[i, sl] = msg_v[i, sl] * sc
                return 0
            lax.fori_loop(0, CB // L, scale, 0)

            pltpu.sync_copy(msg_v, acc_sh.at[sd_v], add=True)
            return 0
        lax.fori_loop(0, nch, chunk, 0)

        plsc.subcore_barrier()
        for off in range(0, RW, CB):
            pltpu.sync_copy(acc_sh.at[pl.ds(s * RW + off, CB), :], msg_v)
            pltpu.sync_copy(msg_v, out.at[c, pl.ds(s * RW + off, CB), :])

    return rgcn(src_arr, dst_arr, edge_type, xw_tab, inv_tab)


def _sc_gated(src_arr, dst_arr, k_tab, qv_tab, NP):
    E = src_arr.shape[0]
    H = k_tab.shape[1]
    EW = E // (NC * NS)
    CC = 40
    DEPTH = 2
    BS = 2000
    nblk = EW // BS
    nit = BS // (CC * DEPTH)
    RW = NP // NS
    CR = CC
    nh = H // L

    @functools.partial(
        pl.kernel,
        out_type=jax.ShapeDtypeStruct((NC, NP, H), f32),
        mesh=_mesh(),
        scratch_types=(
            [pltpu.VMEM((BS,), i32)] +
            [pltpu.VMEM((BS,), i32)] +
            [pltpu.VMEM((CC, H), f32) for _ in range(DEPTH)] +
            [pltpu.VMEM((CC, 2 * H), f32) for _ in range(DEPTH)] +
            [pltpu.VMEM_SHARED((NP, H), f32)] +
            [pltpu.SemaphoreType.DMA for _ in range(2 * DEPTH)]
        ),
    )
    def gated(srca, dsta, kt, qvt, out, *refs):
        s_blk = refs[0]
        d_blk = refs[1]
        kbuf = refs[2:2 + DEPTH]
        qvbuf = refs[2 + DEPTH:2 + 2 * DEPTH]
        iobuf = kbuf[0]
        acc_sh = refs[2 + 2 * DEPTH]
        sems = refs[3 + 2 * DEPTH:]

        c = lax.axis_index("c")
        s = lax.axis_index("s")
        w = c * NS + s

        def zrow(i, _):
            for jj in range(nh):
                iobuf[i, pl.ds(jj * L, L)] = jnp.full((L,), 0.0, f32)
            return 0
        lax.fori_loop(0, CR, zrow, 0)

        for off in range(0, RW, CR):
            pltpu.sync_copy(iobuf, acc_sh.at[pl.ds(s * RW + off, CR), :])
        plsc.subcore_barrier()

        def blk(b, _):
            bbase = w * EW + b * BS
            pltpu.sync_copy(srca.at[pl.ds(bbase, BS)], s_blk)
            pltpu.sync_copy(dsta.at[pl.ds(bbase, BS)], d_blk)

            def body(t, _):
                base0 = t * (CC * DEPTH)
                cps = []
                for k in range(DEPTH):
                    dsl = d_blk.at[pl.ds(base0 + k * CC, CC)]
                    ssl = s_blk.at[pl.ds(base0 + k * CC, CC)]
                    ck = pltpu.async_copy(kt.at[dsl], kbuf[k], sems[2 * k])
                    cq = pltpu.async_copy(qvt.at[ssl], qvbuf[k],
                                          sems[2 * k + 1])
                    cps.append((ck, cq))
                for k in range(DEPTH):
                    cps[k][0].wait()
                    cps[k][1].wait()

                    def gate(i, _):
                        for jj in range(nh):
                            sl = pl.ds(jj * L, L)
                            z = kbuf[k][i, sl] + qvbuf[k][i, sl]
                            sig = 1.0 / (1.0 + jnp.exp(-z))
                            kbuf[k][i, sl] = (
                                sig * qvbuf[k][i, pl.ds(H + jj * L, L)])
                        return 0
                    lax.fori_loop(0, CC, gate, 0)

                    dsl = d_blk.at[pl.ds(base0 + k * CC, CC)]
                    pltpu.sync_copy(kbuf[k], acc_sh.at[dsl], add=True)
                return 0
            lax.fori_loop(0, nit, body, 0)
            return 0
        lax.fori_loop(0, nblk, blk, 0)

        plsc.subcore_barrier()
        for off in range(0, RW, CR):
            pltpu.sync_copy(acc_sh.at[pl.ds(s * RW + off, CR), :], iobuf)
            pltpu.sync_copy(iobuf, out.at[c, pl.ds(s * RW + off, CR), :])

    return gated(src_arr, dst_arr, k_tab, qv_tab)


def kernel(node_features, edge_index, edge_norm, edge_type, comp, basis,
           root, bias1, Wk, bk, Wq, bq, Wv, bv, Wskip, bias2):
    del edge_norm
    N, G = node_features.shape
    R, NB = comp.shape
    H1 = root.shape[1]
    H2 = Wk.shape[1]
    bn = 1000
    NP = 10240

    w2 = _tc_w2(comp, basis.reshape(NB, G * H1))
    xw = _tc_xw(node_features, w2.reshape(R, G, H1), bn)
    xw_tab = xw.reshape(R * N, H1)

    src_arr = edge_index[0]
    dst_arr = edge_index[1]

    cnt = _sc_hist(dst_arr, edge_type, N, R)
    inv3 = _tc_inv(cnt.reshape(NC, (R * N) // G, G))
    inv_tab = inv3.reshape(R * N)

    accs = _sc_rgcn(src_arr, dst_arr, edge_type, xw_tab, inv_tab, N, NP)

    k_tab, qv_tab, skip = _tc_dense2(
        node_features, accs[0, :N], accs[1, :N], root,
        bias1.reshape(1, H1), Wk, bk.reshape(1, H2), Wq, bq.reshape(1, H2),
        Wv, bv.reshape(1, H2), Wskip, bn)

    aggs = _sc_gated(src_arr, dst_arr, k_tab, qv_tab, NP)

    return _tc_final(aggs[0, :N], aggs[1, :N], skip, bias2.reshape(1, H2), bn)


# speedup vs baseline: 7.3435x; 1.0007x over previous
import functools

import jax
import jax.numpy as jnp
from jax import lax
from jax.experimental import pallas as pl
from jax.experimental.pallas import tpu as pltpu
from jax.experimental.pallas import tpu_sc as plsc

NC = 2
NS = 16
L = 16

f32 = jnp.float32
i32 = jnp.int32


def _tc_w2(comp, basis2):
    R, NB = comp.shape
    GH = basis2.shape[1]

    def body(c_ref, b_ref, o_ref):
        o_ref[...] = jnp.dot(c_ref[...], b_ref[...],
                             preferred_element_type=f32)

    return pl.pallas_call(
        body,
        out_shape=jax.ShapeDtypeStruct((R, GH), f32),
    )(comp, basis2)


def _tc_xw(x, w_full, bn):
    N, G = x.shape
    R = w_full.shape[0]
    H1 = w_full.shape[2]
    nb = N // bn

    def body(x_ref, w_ref, o_ref):
        o_ref[0] = jnp.dot(x_ref[...], w_ref[0], preferred_element_type=f32)

    return pl.pallas_call(
        body,
        grid=(R, nb),
        in_specs=[
            pl.BlockSpec((bn, G), lambda r, b: (b, 0)),
            pl.BlockSpec((1, G, H1), lambda r, b: (r, 0, 0)),
        ],
        out_specs=pl.BlockSpec((1, bn, H1), lambda r, b: (r, b, 0)),
        out_shape=jax.ShapeDtypeStruct((R, N, H1), f32),
    )(x, w_full)


def _tc_inv(cnt3):
    _, rows, cols = cnt3.shape

    def body(c_ref, o_ref):
        c = c_ref[0] + c_ref[1]
        o_ref[...] = 1.0 / jnp.maximum(c, 1.0)

    return pl.pallas_call(
        body,
        out_shape=jax.ShapeDtypeStruct((rows, cols), f32),
    )(cnt3)


def _tc_dense2(x, acc0, acc1, root, b1, Wk, bk, Wq, bq, Wv, bv, Wskip, bn):
    N, G = x.shape
    H1 = root.shape[1]
    H2 = Wk.shape[1]
    nb = N // bn

    def body(x_ref, a0_ref, a1_ref, rt_ref, b1_ref, wk_ref, bk_ref,
             wq_ref, bq_ref, wv_ref, bv_ref, ws_ref,
             k_ref, qv_ref, sk_ref):
        out1 = (jnp.dot(x_ref[...], rt_ref[...], preferred_element_type=f32)
                + b1_ref[...] + a0_ref[...] + a1_ref[...])
        k_ref[...] = jnp.dot(out1, wk_ref[...],
                             preferred_element_type=f32) + bk_ref[...]
        qv_ref[:, :H2] = jnp.dot(out1, wq_ref[...],
                                 preferred_element_type=f32) + bq_ref[...]
        qv_ref[:, H2:] = jnp.dot(out1, wv_ref[...],
                                 preferred_element_type=f32) + bv_ref[...]
        sk_ref[...] = jnp.dot(out1, ws_ref[...], preferred_element_type=f32)

    def full(s):
        return pl.BlockSpec(s, lambda b: tuple(0 for _ in s))

    blk = pl.BlockSpec((bn, H1), lambda b: (b, 0))
    return pl.pallas_call(
        body,
        grid=(nb,),
        in_specs=[
            pl.BlockSpec((bn, G), lambda b: (b, 0)),
            blk, blk,
            full((G, H1)), full((1, H1)),
            full((H1, H2)), full((1, H2)),
            full((H1, H2)), full((1, H2)),
            full((H1, H2)), full((1, H2)),
            full((H1, H2)),
        ],
        out_specs=[
            pl.BlockSpec((bn, H2), lambda b: (b, 0)),
            pl.BlockSpec((bn, 2 * H2), lambda b: (b, 0)),
            pl.BlockSpec((bn, H2), lambda b: (b, 0)),
        ],
        out_shape=[
            jax.ShapeDtypeStruct((N, H2), f32),
            jax.ShapeDtypeStruct((N, 2 * H2), f32),
            jax.ShapeDtypeStruct((N, H2), f32),
        ],
    )(x, acc0, acc1, root, b1, Wk, bk, Wq, bq, Wv, bv, Wskip)


def _tc_final(agg0, agg1, skip, b2, bn):
    N, H2 = skip.shape
    nb = N // bn
    blk = pl.BlockSpec((bn, H2), lambda b: (b, 0))

    def body(a0_ref, a1_ref, sk_ref, b2_ref, o_ref):
        o_ref[...] = a0_ref[...] + a1_ref[...] + sk_ref[...] + b2_ref[...]

    return pl.pallas_call(
        body,
        grid=(nb,),
        in_specs=[blk, blk, blk,
                  pl.BlockSpec((1, H2), lambda b: (0, 0))],
        out_specs=blk,
        out_shape=jax.ShapeDtypeStruct((N, H2), f32),
    )(agg0, agg1, skip, b2)


def _mesh():
    return plsc.VectorSubcoreMesh(core_axis_name="c", subcore_axis_name="s",
                                  num_cores=NC, num_subcores=NS)


def _sc_hist(dst_arr, edge_type, N, R):
    E = edge_type.shape[0]
    EW = E // (NC * NS)
    CA = 80
    nch = EW // CA
    T = R * N
    ZW = T // NS

    @functools.partial(
        pl.kernel,
        out_type=jax.ShapeDtypeStruct((NC * T,), f32),
        mesh=_mesh(),
        scratch_types=[
            pltpu.VMEM((EW,), i32),
            pltpu.VMEM((EW,), i32),
            pltpu.VMEM((CA,), i32),
            pltpu.VMEM((CA,), f32),
            pltpu.VMEM((ZW,), f32),
            pltpu.VMEM_SHARED((T,), f32),
            pltpu.SemaphoreType.DMA,
        ],
    )
    def hist(dsta, et, out, t_all, d_all, idx_v, ones_v, zbuf, cnt_sh, sem):
        c = lax.axis_index("c")
        s = lax.axis_index("s")
        w = c * NS + s

        pltpu.sync_copy(et.at[pl.ds(w * EW, EW)], t_all)
        pltpu.sync_copy(dsta.at[pl.ds(w * EW, EW)], d_all)

        def fill1(i, _):
            ones_v[pl.ds(i * L, L)] = jnp.full((L,), 1.0, f32)
            return 0
        lax.fori_loop(0, CA // L, fill1, 0)

        def fill0(i, _):
            zbuf[pl.ds(i * L, L)] = jnp.full((L,), 0.0, f32)
            return 0
        lax.fori_loop(0, ZW // L, fill0, 0)

        pltpu.sync_copy(zbuf, cnt_sh.at[pl.ds(s * ZW, ZW)])
        plsc.subcore_barrier()

        def chunk(j, _):
            base = j * CA

            def cidx(i, _):
                sl = pl.ds(base + i * L, L)
                idx_v[pl.ds(i * L, L)] = t_all[sl] * N + d_all[sl]
                return 0
            lax.fori_loop(0, CA // L, cidx, 0)
            pltpu.sync_copy(ones_v, cnt_sh.at[idx_v], add=True)
            return 0
        lax.fori_loop(0, nch, chunk, 0)

        plsc.subcore_barrier()
        pltpu.sync_copy(cnt_sh.at[pl.ds(s * ZW, ZW)], zbuf)
        pltpu.sync_copy(zbuf, out.at[pl.ds(c * T + s * ZW, ZW)])

    return hist(dst_arr, edge_type)


def _sc_rgcn(src_arr, dst_arr, edge_type, xw_tab, inv_tab, N, NP):
    E = edge_type.shape[0]
    H = xw_tab.shape[1]
    EW = E // (NC * NS)
    CB = 80
    nch = EW // CB
    RW = NP // NS
    nh = H // L

    @functools.partial(
        pl.kernel,
        out_type=jax.ShapeDtypeStruct((NC, NP, H), f32),
        mesh=_mesh(),
        scratch_types=[
            pltpu.VMEM((EW,), i32),
            pltpu.VMEM((EW,), i32),
            pltpu.VMEM((EW,), i32),
            pltpu.VMEM((CB,), i32),
            pltpu.VMEM((CB,), i32),
            pltpu.VMEM((CB,), i32),
            pltpu.VMEM((CB,), f32),
            pltpu.VMEM((CB, H), f32),
            pltpu.VMEM_SHARED((NP, H), f32),
            pltpu.SemaphoreType.DMA,
        ],
    )
    def rgcn(srca, dsta, et, xw, invt, out, t_all, s_all, d_all, im_v, ic_v,
             sd_v, inv_v, msg_v, acc_sh, sem):
        c = lax.axis_index("c")
        s = lax.axis_index("s")
        w = c * NS + s

        pltpu.sync_copy(et.at[pl.ds(w * EW, EW)], t_all)
        pltpu.sync_copy(srca.at[pl.ds(w * EW, EW)], s_all)
        pltpu.sync_copy(dsta.at[pl.ds(w * EW, EW)], d_all)

        def zrow(i, _):
            for jj in range(nh):
                msg_v[i, pl.ds(jj * L, L)] = jnp.full((L,), 0.0, f32)
            return 0
        lax.fori_loop(0, CB, zrow, 0)

        for off in range(0, RW, CB):
            pltpu.sync_copy(msg_v, acc_sh.at[pl.ds(s * RW + off, CB), :])
        plsc.subcore_barrier()

        def chunk(j, _):
            base = j * CB

            def cidx(i, _):
                sl = pl.ds(base + i * L, L)
                ol = pl.ds(i * L, L)
                tt = t_all[sl] * N
                im_v[ol] = tt + s_all[sl]
                ic_v[ol] = tt + d_all[sl]
                sd_v[ol] = d_all[sl]
                return 0
            lax.fori_loop(0, CB // L, cidx, 0)

            cm = pltpu.async_copy(xw.at[im_v], msg_v, sem)
            ci = pltpu.async_copy(invt.at[ic_v], inv_v, sem)
            cm.wait()
            ci.wait()

            def scale(g, _):
                iv = inv_v[pl.ds(g * L, L)]
                for e in range(L):
                    i = g * L + e
                    sc = iv[e]
                    for jj in range(nh):
                        sl = pl.ds(jj * L, L)
                        msg_v[i, sl] = msg_v[i, sl] * sc
                return 0
            lax.fori_loop(0, CB // L, scale, 0)

            pltpu.sync_copy(msg_v, acc_sh.at[sd_v], add=True)
            return 0
        lax.fori_loop(0, nch, chunk, 0)

        plsc.subcore_barrier()
        for off in range(0, RW, CB):
            pltpu.sync_copy(acc_sh.at[pl.ds(s * RW + off, CB), :], msg_v)
            pltpu.sync_copy(msg_v, out.at[c, pl.ds(s * RW + off, CB), :])

    return rgcn(src_arr, dst_arr, edge_type, xw_tab, inv_tab)


def _sc_gated(src_arr, dst_arr, k_tab, qv_tab, NP):
    E = src_arr.shape[0]
    H = k_tab.shape[1]
    EW = E // (NC * NS)
    CC = 40
    DEPTH = 2
    BS = 2000
    nblk = EW // BS
    nit = BS // (CC * DEPTH)
    RW = NP // NS
    CR = CC
    nh = H // L

    @functools.partial(
        pl.kernel,
        out_type=jax.ShapeDtypeStruct((NC, NP, H), f32),
        mesh=_mesh(),
        scratch_types=(
            [pltpu.VMEM((BS,), i32)] +
            [pltpu.VMEM((BS,), i32)] +
            [pltpu.VMEM((CC, H), f32) for _ in range(DEPTH)] +
            [pltpu.VMEM((CC, 2 * H), f32) for _ in range(DEPTH)] +
            [pltpu.VMEM_SHARED((NP, H), f32)] +
            [pltpu.SemaphoreType.DMA for _ in range(2 * DEPTH)]
        ),
    )
    def gated(srca, dsta, kt, qvt, out, *refs):
        s_blk = refs[0]
        d_blk = refs[1]
        kbuf = refs[2:2 + DEPTH]
        qvbuf = refs[2 + DEPTH:2 + 2 * DEPTH]
        iobuf = kbuf[0]
        acc_sh = refs[2 + 2 * DEPTH]
        sems = refs[3 + 2 * DEPTH:]

        c = lax.axis_index("c")
        s = lax.axis_index("s")
        w = c * NS + s

        def zrow(i, _):
            for jj in range(nh):
                iobuf[i, pl.ds(jj * L, L)] = jnp.full((L,), 0.0, f32)
            return 0
        lax.fori_loop(0, CR, zrow, 0)

        for off in range(0, RW, CR):
            pltpu.sync_copy(iobuf, acc_sh.at[pl.ds(s * RW + off, CR), :])
        plsc.subcore_barrier()

        def blk(b, _):
            bbase = w * EW + b * BS
            pltpu.sync_copy(srca.at[pl.ds(bbase, BS)], s_blk)
            pltpu.sync_copy(dsta.at[pl.ds(bbase, BS)], d_blk)

            def body(t, _):
                base0 = t * (CC * DEPTH)
                cps = []
                for k in range(DEPTH):
                    dsl = d_blk.at[pl.ds(base0 + k * CC, CC)]
                    ssl = s_blk.at[pl.ds(base0 + k * CC, CC)]
                    ck = pltpu.async_copy(kt.at[dsl], kbuf[k], sems[2 * k])
                    cq = pltpu.async_copy(qvt.at[ssl], qvbuf[k],
                                          sems[2 * k + 1])
                    cps.append((ck, cq))
                for k in range(DEPTH):
                    cps[k][0].wait()
                    cps[k][1].wait()

                    def gate(i, _):
                        for jj in range(nh):
                            sl = pl.ds(jj * L, L)
                            z = kbuf[k][i, sl] + qvbuf[k][i, sl]
                            den = 1.0 + jnp.exp(-z)
                            kbuf[k][i, sl] = (
                                qvbuf[k][i, pl.ds(H + jj * L, L)] / den)
                        return 0
                    lax.fori_loop(0, CC, gate, 0)

                    dsl = d_blk.at[pl.ds(base0 + k * CC, CC)]
                    pltpu.sync_copy(kbuf[k], acc_sh.at[dsl], add=True)
                return 0
            lax.fori_loop(0, nit, body, 0)
            return 0
        lax.fori_loop(0, nblk, blk, 0)

        plsc.subcore_barrier()
        for off in range(0, RW, CR):
            pltpu.sync_copy(acc_sh.at[pl.ds(s * RW + off, CR), :], iobuf)
            pltpu.sync_copy(iobuf, out.at[c, pl.ds(s * RW + off, CR), :])

    return gated(src_arr, dst_arr, k_tab, qv_tab)


def kernel(node_features, edge_index, edge_norm, edge_type, comp, basis,
           root, bias1, Wk, bk, Wq, bq, Wv, bv, Wskip, bias2):
    del edge_norm
    N, G = node_features.shape
    R, NB = comp.shape
    H1 = root.shape[1]
    H2 = Wk.shape[1]
    bn = 1000
    NP = 10240

    w2 = _tc_w2(comp, basis.reshape(NB, G * H1))
    xw = _tc_xw(node_features, w2.reshape(R, G, H1), bn)
    xw_tab = xw.reshape(R * N, H1)

    src_arr = edge_index[0]
    dst_arr = edge_index[1]

    cnt = _sc_hist(dst_arr, edge_type, N, R)
    inv3 = _tc_inv(cnt.reshape(NC, (R * N) // G, G))
    inv_tab = inv3.reshape(R * N)

    accs = _sc_rgcn(src_arr, dst_arr, edge_type, xw_tab, inv_tab, N, NP)

    k_tab, qv_tab, skip = _tc_dense2(
        node_features, accs[0, :N], accs[1, :N], root,
        bias1.reshape(1, H1), Wk, bk.reshape(1, H2), Wq, bq.reshape(1, H2),
        Wv, bv.reshape(1, H2), Wskip, bn)

    aggs = _sc_gated(src_arr, dst_arr, k_tab, qv_tab, NP)

    return _tc_final(aggs[0, :N], aggs[1, :N], skip, bias2.reshape(1, H2), bn)


# speedup vs baseline: 17.5547x; 2.3905x over previous
import functools

import jax
import jax.numpy as jnp
from jax import lax
from jax.experimental import pallas as pl
from jax.experimental.pallas import tpu as pltpu
from jax.experimental.pallas import tpu_sc as plsc

NC = 2
NS = 16
L = 16

f32 = jnp.float32
i32 = jnp.int32


def _tc_w2(comp, basis2):
    R, NB = comp.shape
    GH = basis2.shape[1]

    def body(c_ref, b_ref, o_ref):
        o_ref[...] = jnp.dot(c_ref[...], b_ref[...],
                             preferred_element_type=f32)

    return pl.pallas_call(
        body,
        out_shape=jax.ShapeDtypeStruct((R, GH), f32),
    )(comp, basis2)


def _tc_xw(x, w_full, bn):
    N, G = x.shape
    R = w_full.shape[0]
    H1 = w_full.shape[2]
    nb = N // bn

    def body(x_ref, w_ref, o_ref):
        o_ref[0] = jnp.dot(x_ref[...], w_ref[0], preferred_element_type=f32)

    return pl.pallas_call(
        body,
        grid=(R, nb),
        in_specs=[
            pl.BlockSpec((bn, G), lambda r, b: (b, 0)),
            pl.BlockSpec((1, G, H1), lambda r, b: (r, 0, 0)),
        ],
        out_specs=pl.BlockSpec((1, bn, H1), lambda r, b: (r, b, 0)),
        out_shape=jax.ShapeDtypeStruct((R, N, H1), f32),
    )(x, w_full)


def _tc_inv(cnt3):
    _, rows, cols = cnt3.shape

    def body(c_ref, o_ref):
        c = c_ref[0] + c_ref[1]
        o_ref[...] = 1.0 / jnp.maximum(c, 1.0)

    return pl.pallas_call(
        body,
        out_shape=jax.ShapeDtypeStruct((rows, cols), f32),
    )(cnt3)


def _tc_dense2(x, acc0, acc1, root, b1, Wk, bk, Wq, bq, Wv, bv, Wskip, bn):
    N, G = x.shape
    H1 = root.shape[1]
    H2 = Wk.shape[1]
    nb = N // bn

    def body(x_ref, a0_ref, a1_ref, rt_ref, b1_ref, wk_ref, bk_ref,
             wq_ref, bq_ref, wv_ref, bv_ref, ws_ref,
             k_ref, qv_ref, sk_ref):
        out1 = (jnp.dot(x_ref[...], rt_ref[...], preferred_element_type=f32)
                + b1_ref[...] + a0_ref[...] + a1_ref[...])
        k_ref[...] = jnp.dot(out1, wk_ref[...],
                             preferred_element_type=f32) + bk_ref[...]
        qv_ref[:, :H2] = jnp.dot(out1, wq_ref[...],
                                 preferred_element_type=f32) + bq_ref[...]
        qv_ref[:, H2:] = jnp.dot(out1, wv_ref[...],
                                 preferred_element_type=f32) + bv_ref[...]
        sk_ref[...] = jnp.dot(out1, ws_ref[...], preferred_element_type=f32)

    def full(s):
        return pl.BlockSpec(s, lambda b: tuple(0 for _ in s))

    blk = pl.BlockSpec((bn, H1), lambda b: (b, 0))
    return pl.pallas_call(
        body,
        grid=(nb,),
        in_specs=[
            pl.BlockSpec((bn, G), lambda b: (b, 0)),
            blk, blk,
            full((G, H1)), full((1, H1)),
            full((H1, H2)), full((1, H2)),
            full((H1, H2)), full((1, H2)),
            full((H1, H2)), full((1, H2)),
            full((H1, H2)),
        ],
        out_specs=[
            pl.BlockSpec((bn, H2), lambda b: (b, 0)),
            pl.BlockSpec((bn, 2 * H2), lambda b: (b, 0)),
            pl.BlockSpec((bn, H2), lambda b: (b, 0)),
        ],
        out_shape=[
            jax.ShapeDtypeStruct((N, H2), f32),
            jax.ShapeDtypeStruct((N, 2 * H2), f32),
            jax.ShapeDtypeStruct((N, H2), f32),
        ],
    )(x, acc0, acc1, root, b1, Wk, bk, Wq, bq, Wv, bv, Wskip)


def _tc_final(agg0, agg1, skip, b2, bn):
    N, H2 = skip.shape
    nb = N // bn
    blk = pl.BlockSpec((bn, H2), lambda b: (b, 0))

    def body(a0_ref, a1_ref, sk_ref, b2_ref, o_ref):
        o_ref[...] = a0_ref[...] + a1_ref[...] + sk_ref[...] + b2_ref[...]

    return pl.pallas_call(
        body,
        grid=(nb,),
        in_specs=[blk, blk, blk,
                  pl.BlockSpec((1, H2), lambda b: (0, 0))],
        out_specs=blk,
        out_shape=jax.ShapeDtypeStruct((N, H2), f32),
    )(agg0, agg1, skip, b2)


def _mesh():
    return plsc.VectorSubcoreMesh(core_axis_name="c", subcore_axis_name="s",
                                  num_cores=NC, num_subcores=NS)


def _sc_hist(dst_arr, edge_type, N, R):
    E = edge_type.shape[0]
    EW = E // (NC * NS)
    CA = 80
    nch = EW // CA
    T = R * N
    ZW = T // NS

    @functools.partial(
        pl.kernel,
        out_type=jax.ShapeDtypeStruct((NC * T,), f32),
        mesh=_mesh(),
        scratch_types=[
            pltpu.VMEM((EW,), i32),
            pltpu.VMEM((EW,), i32),
            pltpu.VMEM((CA,), i32),
            pltpu.VMEM((CA,), f32),
            pltpu.VMEM((ZW,), f32),
            pltpu.VMEM_SHARED((T,), f32),
            pltpu.SemaphoreType.DMA,
        ],
    )
    def hist(dsta, et, out, t_all, d_all, idx_v, ones_v, zbuf, cnt_sh, sem):
        c = lax.axis_index("c")
        s = lax.axis_index("s")
        w = c * NS + s

        pltpu.sync_copy(et.at[pl.ds(w * EW, EW)], t_all)
        pltpu.sync_copy(dsta.at[pl.ds(w * EW, EW)], d_all)

        def fill1(i, _):
            ones_v[pl.ds(i * L, L)] = jnp.full((L,), 1.0, f32)
            return 0
        lax.fori_loop(0, CA // L, fill1, 0)

        def fill0(i, _):
            zbuf[pl.ds(i * L, L)] = jnp.full((L,), 0.0, f32)
            return 0
        lax.fori_loop(0, ZW // L, fill0, 0)

        pltpu.sync_copy(zbuf, cnt_sh.at[pl.ds(s * ZW, ZW)])
        plsc.subcore_barrier()

        def chunk(j, _):
            base = j * CA

            def cidx(i, _):
                sl = pl.ds(base + i * L, L)
                idx_v[pl.ds(i * L, L)] = t_all[sl] * N + d_all[sl]
                return 0
            lax.fori_loop(0, CA // L, cidx, 0)
            pltpu.sync_copy(ones_v, cnt_sh.at[idx_v], add=True)
            return 0
        lax.fori_loop(0, nch, chunk, 0)

        plsc.subcore_barrier()
        pltpu.sync_copy(cnt_sh.at[pl.ds(s * ZW, ZW)], zbuf)
        pltpu.sync_copy(zbuf, out.at[pl.ds(c * T + s * ZW, ZW)])

    return hist(dst_arr, edge_type)


def _sc_rgcn(src_arr, dst_arr, edge_type, xw_tab, inv_tab, N, NP):
    E = edge_type.shape[0]
    H = xw_tab.shape[1]
    EW = E // (NC * NS)
    CB = 80
    nch = EW // CB
    RW = NP // NS
    nh = H // L

    @functools.partial(
        pl.kernel,
        out_type=jax.ShapeDtypeStruct((NC, NP, H), f32),
        mesh=_mesh(),
        scratch_types=[
            pltpu.VMEM((EW,), i32),
            pltpu.VMEM((EW,), i32),
            pltpu.VMEM((EW,), i32),
            pltpu.VMEM((CB,), i32),
            pltpu.VMEM((CB,), i32),
            pltpu.VMEM((CB,), i32),
            pltpu.VMEM((CB,), f32),
            pltpu.VMEM((CB, H), f32),
            pltpu.VMEM_SHARED((NP, H), f32),
            pltpu.SemaphoreType.DMA,
        ],
    )
    def rgcn(srca, dsta, et, xw, invt, out, t_all, s_all, d_all, im_v, ic_v,
             sd_v, inv_v, msg_v, acc_sh, sem):
        c = lax.axis_index("c")
        s = lax.axis_index("s")
        w = c * NS + s

        pltpu.sync_copy(et.at[pl.ds(w * EW, EW)], t_all)
        pltpu.sync_copy(srca.at[pl.ds(w * EW, EW)], s_all)
        pltpu.sync_copy(dsta.at[pl.ds(w * EW, EW)], d_all)

        def zrow(i, _):
            for jj in range(nh):
                msg_v[i, pl.ds(jj * L, L)] = jnp.full((L,), 0.0, f32)
            return 0
        lax.fori_loop(0, CB, zrow, 0)

        for off in range(0, RW, CB):
            pltpu.sync_copy(msg_v, acc_sh.at[pl.ds(s * RW + off, CB), :])
        plsc.subcore_barrier()

        def chunk(j, _):
            base = j * CB

            def cidx(i, _):
                sl = pl.ds(base + i * L, L)
                ol = pl.ds(i * L, L)
                tt = t_all[sl] * N
                im_v[ol] = tt + s_all[sl]
                ic_v[ol] = tt + d_all[sl]
                sd_v[ol] = d_all[sl]
                return 0
            lax.fori_loop(0, CB // L, cidx, 0)

            cm = pltpu.async_copy(xw.at[im_v], msg_v, sem)
            ci = pltpu.async_copy(invt.at[ic_v], inv_v, sem)
            cm.wait()
            ci.wait()

            def scale(g, _):
                iv = inv_v[pl.ds(g * L, L)]
                for e in range(L):
                    i = g * L + e
                    sc = iv[e]
                    for jj in range(nh):
                        sl = pl.ds(jj * L, L)
                        msg_v[i, sl] = msg_v[i, sl] * sc
                return 0
            lax.fori_loop(0, CB // L, scale, 0)

            pltpu.sync_copy(msg_v, acc_sh.at[sd_v], add=True)
            return 0
        lax.fori_loop(0, nch, chunk, 0)

        plsc.subcore_barrier()
        for off in range(0, RW, CB):
            pltpu.sync_copy(acc_sh.at[pl.ds(s * RW + off, CB), :], msg_v)
            pltpu.sync_copy(msg_v, out.at[c, pl.ds(s * RW + off, CB), :])

    return rgcn(src_arr, dst_arr, edge_type, xw_tab, inv_tab)


def _sc_gated(src_arr, dst_arr, k_tab, qv_tab, NP):
    E = src_arr.shape[0]
    H = k_tab.shape[1]
    EW = E // (NC * NS)
    CC = 40
    DEPTH = 2
    BS = 2000
    nblk = EW // BS
    nit = BS // (CC * DEPTH)
    RW = NP // NS
    CR = CC
    nh = H // L

    @functools.partial(
        pl.kernel,
        out_type=jax.ShapeDtypeStruct((NC, NP, H), f32),
        mesh=_mesh(),
        scratch_types=(
            [pltpu.VMEM((BS,), i32)] +
            [pltpu.VMEM((BS,), i32)] +
            [pltpu.VMEM((CC, H), f32) for _ in range(DEPTH)] +
            [pltpu.VMEM((CC, 2 * H), f32) for _ in range(DEPTH)] +
            [pltpu.VMEM_SHARED((NP, H), f32)] +
            [pltpu.SemaphoreType.DMA for _ in range(2 * DEPTH)]
        ),
    )
    def gated(srca, dsta, kt, qvt, out, *refs):
        s_blk = refs[0]
        d_blk = refs[1]
        kbuf = refs[2:2 + DEPTH]
        qvbuf = refs[2 + DEPTH:2 + 2 * DEPTH]
        iobuf = kbuf[0]
        acc_sh = refs[2 + 2 * DEPTH]
        sems = refs[3 + 2 * DEPTH:]

        c = lax.axis_index("c")
        s = lax.axis_index("s")
        w = c * NS + s

        def zrow(i, _):
            for jj in range(nh):
                iobuf[i, pl.ds(jj * L, L)] = jnp.full((L,), 0.0, f32)
            return 0
        lax.fori_loop(0, CR, zrow, 0)

        for off in range(0, RW, CR):
            pltpu.sync_copy(iobuf, acc_sh.at[pl.ds(s * RW + off, CR), :])
        plsc.subcore_barrier()

        def blk(b, _):
            bbase = w * EW + b * BS
            pltpu.sync_copy(srca.at[pl.ds(bbase, BS)], s_blk)
            pltpu.sync_copy(dsta.at[pl.ds(bbase, BS)], d_blk)

            def body(t, _):
                base0 = t * (CC * DEPTH)
                cps = []
                for k in range(DEPTH):
                    dsl = d_blk.at[pl.ds(base0 + k * CC, CC)]
                    ssl = s_blk.at[pl.ds(base0 + k * CC, CC)]
                    ck = pltpu.async_copy(kt.at[dsl], kbuf[k], sems[2 * k])
                    cq = pltpu.async_copy(qvt.at[ssl], qvbuf[k],
                                          sems[2 * k + 1])
                    cps.append((ck, cq))
                for k in range(DEPTH):
                    cps[k][0].wait()
                    cps[k][1].wait()

                    def gate(i, _):
                        zs = [kbuf[k][i, pl.ds(jj * L, L)]
                              + qvbuf[k][i, pl.ds(jj * L, L)]
                              for jj in range(nh)]
                        vs = [qvbuf[k][i, pl.ds(H + jj * L, L)]
                              for jj in range(nh)]
                        es = [jnp.exp(-z) for z in zs]
                        res = [v / (1.0 + e) for v, e in zip(vs, es)]
                        for jj in range(nh):
                            kbuf[k][i, pl.ds(jj * L, L)] = res[jj]
                        return 0
                    lax.fori_loop(0, CC, gate, 0)

                    dsl = d_blk.at[pl.ds(base0 + k * CC, CC)]
                    pltpu.sync_copy(kbuf[k], acc_sh.at[dsl], add=True)
                return 0
            lax.fori_loop(0, nit, body, 0)
            return 0
        lax.fori_loop(0, nblk, blk, 0)

        plsc.subcore_barrier()
        for off in range(0, RW, CR):
            pltpu.sync_copy(acc_sh.at[pl.ds(s * RW + off, CR), :], iobuf)
            pltpu.sync_copy(iobuf, out.at[c, pl.ds(s * RW + off, CR), :])

    return gated(src_arr, dst_arr, k_tab, qv_tab)


def kernel(node_features, edge_index, edge_norm, edge_type, comp, basis,
           root, bias1, Wk, bk, Wq, bq, Wv, bv, Wskip, bias2):
    del edge_norm
    N, G = node_features.shape
    R, NB = comp.shape
    H1 = root.shape[1]
    H2 = Wk.shape[1]
    bn = 1000
    NP = 10240

    w2 = _tc_w2(comp, basis.reshape(NB, G * H1))
    xw = _tc_xw(node_features, w2.reshape(R, G, H1), bn)
    xw_tab = xw.reshape(R * N, H1)

    src_arr = edge_index[0]
    dst_arr = edge_index[1]

    cnt = _sc_hist(dst_arr, edge_type, N, R)
    inv3 = _tc_inv(cnt.reshape(NC, (R * N) // G, G))
    inv_tab = inv3.reshape(R * N)

    accs = _sc_rgcn(src_arr, dst_arr, edge_type, xw_tab, inv_tab, N, NP)

    k_tab, qv_tab, skip = _tc_dense2(
        node_features, accs[0, :N], accs[1, :N], root,
        bias1.reshape(1, H1), Wk, bk.reshape(1, H2), Wq, bq.reshape(1, H2),
        Wv, bv.reshape(1, H2), Wskip, bn)

    aggs = _sc_gated(src_arr, dst_arr, k_tab, qv_tab, NP)

    return _tc_final(aggs[0, :N], aggs[1, :N], skip, bias2.reshape(1, H2), bn)
